# baseline (speedup 1.0000x reference)
import functools
import math

import jax
import jax.numpy as jnp
import numpy as np
from jax import lax
from jax.experimental import pallas as pl
from jax.experimental.pallas import tpu as pltpu

try:
    from jax.experimental.pallas import tpu_sc as plsc
    _HAS_SC = True
except ImportError:
    _HAS_SC = False

B, L, D, H, NL = 8, 512, 128, 8, 4
V = 100000
NET = 16
NC = 2
HD = D // H


def _pos_emb_np(length, dim):
    pos = np.arange(length)[:, None].astype(np.float64)
    i = np.arange(dim // 2)[None, :].astype(np.float64)
    ang = pos / np.power(10000.0, 2.0 * i / dim)
    pe = np.zeros((length, dim), dtype=np.float32)
    pe[:, 0::2] = np.sin(ang)
    pe[:, 1::2] = np.cos(ang)
    return pe


def _sc_gather(word_emb, flat_ids):
    info = plsc.get_sparse_core_info()
    ncores, nsub = info.num_cores, info.num_subcores
    nw = ncores * nsub
    total = B * L
    per_w = total // nw

    mesh = plsc.VectorSubcoreMesh(core_axis_name="c", subcore_axis_name="s")

    @functools.partial(
        pl.kernel,
        mesh=mesh,
        out_type=jax.ShapeDtypeStruct((total, D), jnp.float32),
        scratch_types=[
            pltpu.VMEM((per_w,), jnp.int32),
            pltpu.VMEM((per_w, D), jnp.float32),
            pltpu.SemaphoreType.DMA,
        ],
    )
    def gather_kernel(table_hbm, idx_hbm, out_hbm, idx_v, rows_v, sem):
        wid = lax.axis_index("s") * ncores + lax.axis_index("c")
        base = wid * per_w
        pltpu.sync_copy(idx_hbm.at[pl.ds(base, per_w)], idx_v)
        pltpu.async_copy(table_hbm.at[idx_v], rows_v, sem).wait()
        pltpu.sync_copy(rows_v, out_hbm.at[pl.ds(base, per_w)])

    return gather_kernel(word_emb, flat_ids)


BB = 2


def _gat_kernel(h0_ref, adj_ref, et_ref, wall_ref, wo_ref, lns_ref,
                lnb_ref, wcls_ref, cmat_ref, jmat_ref, out_ref,
                h_scr, o_scr):
    neg_col = jnp.full((L, HD), -1e9, jnp.float32)
    ones_col = jnp.ones((L, HD), jnp.float32)
    etms = []
    for bb in range(BB):
        h_scr[bb] = h0_ref[bb]
        adj32 = adj_ref[bb].astype(jnp.int32)
        etms.append(NET + adj32 * (et_ref[bb] - NET))

    for l in range(NL):
        hs = [h_scr[bb] for bb in range(BB)]
        qkvps = [jnp.dot(hs[bb], wall_ref[l],
                         preferred_element_type=jnp.float32)
                 for bb in range(BB)]
        for hh in range(H):
            for bb in range(BB):
                qkvp = qkvps[bb]
                qh = qkvp[:, hh * HD:(hh + 1) * HD]
                kh = qkvp[:, D + hh * HD:D + (hh + 1) * HD]
                vh = qkvp[:, 2 * D + hh * HD:2 * D + (hh + 1) * HD]
                ph = qkvp[:, 3 * D + hh * HD:3 * D + (hh + 1) * HD]
                pf = jnp.concatenate([ph, neg_col], axis=1)
                scores = jax.lax.dot_general(
                    qh, kh, (((1,), (1,)), ((), ())),
                    preferred_element_type=jnp.float32)
                scores += jnp.take_along_axis(pf, etms[bb], axis=1)
                e = jnp.exp(scores)
                va = jnp.concatenate([vh, ones_col], axis=1)
                oa = jnp.dot(e, va, preferred_element_type=jnp.float32)
                o_scr[bb, :, hh * HD:(hh + 1) * HD] = \
                    oa[:, :HD] / oa[:, HD:2 * HD]
        for bb in range(BB):
            proj = jnp.dot(o_scr[bb], wo_ref[l],
                           preferred_element_type=jnp.float32)
            x = hs[bb] + proj
            xc = jnp.dot(x, cmat_ref[...],
                         preferred_element_type=jnp.float32)
            v2 = jnp.dot(xc * xc, jmat_ref[...],
                         preferred_element_type=jnp.float32)
            h_scr[bb] = xc * jax.lax.rsqrt(v2 + 1e-5) * lns_ref[l] \
                + lnb_ref[l]

    for bb in range(BB):
        cls_h = h_scr[bb, 0:1, :]
        out_ref[bb] = jnp.dot(cls_h, wcls_ref[...],
                              preferred_element_type=jnp.float32)


def _run_gat(h0, adj_i8, edge_types, W_all, Wo,
             ln_scale, ln_bias, Wcls_pad):
    rep2 = lambda b: (0, 0)
    rep3 = lambda b: (0, 0, 0)

    in_specs = [
            pl.BlockSpec((BB, L, D), lambda b: (b, 0, 0)),
            pl.BlockSpec((BB, L, L), lambda b: (b, 0, 0)),
            pl.BlockSpec((BB, L, L), lambda b: (b, 0, 0)),
            pl.BlockSpec((NL, D, 4 * D), rep3),
            pl.BlockSpec((NL, D, D), rep3),
            pl.BlockSpec((NL, D), rep2),
            pl.BlockSpec((NL, D), rep2),
            pl.BlockSpec((D, 128), rep2),
            pl.BlockSpec((D, D), rep2),
            pl.BlockSpec((D, D), rep2),
    ]
    cmat = jnp.eye(D, dtype=jnp.float32) - 1.0 / D
    jmat = jnp.full((D, D), 1.0 / D, jnp.float32)
    return pl.pallas_call(
        _gat_kernel,
        grid=(B // BB,),
        in_specs=in_specs,
        out_specs=pl.BlockSpec((BB, 1, 128), lambda b: (b, 0, 0)),
        out_shape=jax.ShapeDtypeStruct((B, 1, 128), jnp.float32),
        scratch_shapes=[
            pltpu.VMEM((BB, L, D), jnp.float32),
            pltpu.VMEM((BB, L, D), jnp.float32),
        ],
    )(h0, adj_i8, edge_types, W_all, Wo,
      ln_scale, ln_bias, Wcls_pad, cmat, jmat)


def kernel(word_ids, batched_adj, edge_types, word_emb, edge_emb,
           Wq, Wk, Wv, Wo, ln_scale, ln_bias, W_cls, b_cls):
    flat_ids = word_ids.reshape(B * L).astype(jnp.int32)
    rows = _sc_gather(word_emb, flat_ids)
    pos = jnp.asarray(_pos_emb_np(L, D))
    h0 = rows.reshape(B, L, D) + pos[None]

    adj_i8 = batched_adj.astype(jnp.int8)
    Wcls_pad = jnp.zeros((D, 128), jnp.float32).at[:, :NC].set(W_cls)
    eeT = edge_emb[:HD].T
    ee_blk = jax.scipy.linalg.block_diag(*([eeT] * H)).astype(jnp.float32)
    inv = jnp.float32(1.0 / math.sqrt(HD))
    W_all = jnp.concatenate(
        [Wq * inv, Wk, Wv, jnp.einsum('lde,ef->ldf', Wq, ee_blk) * inv],
        axis=2)

    out = _run_gat(h0, adj_i8, edge_types.astype(jnp.int32), W_all,
                   Wo, ln_scale, ln_bias, Wcls_pad)
    return out.reshape(B, 128)[:, :NC] + b_cls[None]


# speedup vs baseline: 1.1378x; 1.1378x over previous
import functools
import math

import jax
import jax.numpy as jnp
import numpy as np
from jax import lax
from jax.experimental import pallas as pl
from jax.experimental.pallas import tpu as pltpu

try:
    from jax.experimental.pallas import tpu_sc as plsc
    _HAS_SC = True
except ImportError:
    _HAS_SC = False

B, L, D, H, NL = 8, 512, 128, 8, 4
V = 100000
NET = 16
NC = 2
HD = D // H


def _pos_emb_np(length, dim):
    pos = np.arange(length)[:, None].astype(np.float64)
    i = np.arange(dim // 2)[None, :].astype(np.float64)
    ang = pos / np.power(10000.0, 2.0 * i / dim)
    pe = np.zeros((length, dim), dtype=np.float32)
    pe[:, 0::2] = np.sin(ang)
    pe[:, 1::2] = np.cos(ang)
    return pe


def _sc_gather(word_emb, flat_ids):
    info = plsc.get_sparse_core_info()
    ncores, nsub = info.num_cores, info.num_subcores
    nw = ncores * nsub
    total = B * L
    per_w = total // nw

    mesh = plsc.VectorSubcoreMesh(core_axis_name="c", subcore_axis_name="s")

    @functools.partial(
        pl.kernel,
        mesh=mesh,
        out_type=jax.ShapeDtypeStruct((total, D), jnp.float32),
        scratch_types=[
            pltpu.VMEM((per_w,), jnp.int32),
            pltpu.VMEM((per_w, D), jnp.float32),
            pltpu.SemaphoreType.DMA,
        ],
    )
    def gather_kernel(table_hbm, idx_hbm, out_hbm, idx_v, rows_v, sem):
        wid = lax.axis_index("s") * ncores + lax.axis_index("c")
        base = wid * per_w
        pltpu.sync_copy(idx_hbm.at[pl.ds(base, per_w)], idx_v)
        pltpu.async_copy(table_hbm.at[idx_v], rows_v, sem).wait()
        pltpu.sync_copy(rows_v, out_hbm.at[pl.ds(base, per_w)])

    return gather_kernel(word_emb, flat_ids)


def _gat_kernel(h0_ref, adj_ref, et_ref, wall_ref, wo_ref, lns_ref,
                lnb_ref, wcls_ref, cmat_ref, jmat_ref, out_ref,
                h_scr, o_scr):
    h_scr[...] = h0_ref[0]
    adj32 = adj_ref[0].astype(jnp.int32)
    etm = NET + adj32 * (et_ref[0] - NET)
    neg_col = jnp.full((L, HD), -1e9, jnp.float32)
    ones_col = jnp.ones((L, HD), jnp.float32)

    for l in range(NL):
        h = h_scr[...]
        qkvp = jnp.dot(h, wall_ref[l], preferred_element_type=jnp.float32)
        for hh in range(H):
            qh = qkvp[:, hh * HD:(hh + 1) * HD]
            kh = qkvp[:, D + hh * HD:D + (hh + 1) * HD]
            vh = qkvp[:, 2 * D + hh * HD:2 * D + (hh + 1) * HD]
            ph = qkvp[:, 3 * D + hh * HD:3 * D + (hh + 1) * HD]
            pf = jnp.concatenate([ph, neg_col], axis=1)
            scores = jax.lax.dot_general(
                qh.astype(jnp.bfloat16), kh.astype(jnp.bfloat16),
                (((1,), (1,)), ((), ())),
                preferred_element_type=jnp.float32)
            scores += jnp.take_along_axis(pf, etm, axis=1)
            e = jnp.exp(scores)
            va = jnp.concatenate([vh, ones_col], axis=1)
            oa = jnp.dot(e.astype(jnp.bfloat16), va.astype(jnp.bfloat16),
                         preferred_element_type=jnp.float32)
            o_scr[:, hh * HD:(hh + 1) * HD] = oa[:, :HD] / oa[:, HD:2 * HD]
        proj = jnp.dot(o_scr[...], wo_ref[l],
                       preferred_element_type=jnp.float32)
        x = h + proj
        xc = jnp.dot(x, cmat_ref[...], preferred_element_type=jnp.float32)
        v2 = jnp.dot(xc * xc, jmat_ref[...],
                     preferred_element_type=jnp.float32)
        h_scr[...] = xc * jax.lax.rsqrt(v2 + 1e-5) * lns_ref[l] \
            + lnb_ref[l]

    cls_h = h_scr[0:1, :]
    out_ref[0] = jnp.dot(cls_h, wcls_ref[...],
                         preferred_element_type=jnp.float32)


def _run_gat(h0, adj_i8, edge_types, W_all, Wo,
             ln_scale, ln_bias, Wcls_pad):
    rep2 = lambda b: (0, 0)
    rep3 = lambda b: (0, 0, 0)

    in_specs = [
            pl.BlockSpec((1, L, D), lambda b: (b, 0, 0)),
            pl.BlockSpec((1, L, L), lambda b: (b, 0, 0)),
            pl.BlockSpec((1, L, L), lambda b: (b, 0, 0)),
            pl.BlockSpec((NL, D, 4 * D), rep3),
            pl.BlockSpec((NL, D, D), rep3),
            pl.BlockSpec((NL, D), rep2),
            pl.BlockSpec((NL, D), rep2),
            pl.BlockSpec((D, 128), rep2),
            pl.BlockSpec((D, D), rep2),
            pl.BlockSpec((D, D), rep2),
    ]
    cmat = jnp.eye(D, dtype=jnp.float32) - 1.0 / D
    jmat = jnp.full((D, D), 1.0 / D, jnp.float32)
    return pl.pallas_call(
        _gat_kernel,
        grid=(B,),
        in_specs=in_specs,
        out_specs=pl.BlockSpec((1, 1, 128), lambda b: (b, 0, 0)),
        out_shape=jax.ShapeDtypeStruct((B, 1, 128), jnp.float32),
        scratch_shapes=[
            pltpu.VMEM((L, D), jnp.float32),
            pltpu.VMEM((L, D), jnp.float32),
        ],
    )(h0, adj_i8, edge_types, W_all, Wo,
      ln_scale, ln_bias, Wcls_pad, cmat, jmat)


def kernel(word_ids, batched_adj, edge_types, word_emb, edge_emb,
           Wq, Wk, Wv, Wo, ln_scale, ln_bias, W_cls, b_cls):
    flat_ids = word_ids.reshape(B * L).astype(jnp.int32)
    rows = _sc_gather(word_emb, flat_ids)
    pos = jnp.asarray(_pos_emb_np(L, D))
    h0 = rows.reshape(B, L, D) + pos[None]

    adj_i8 = batched_adj.astype(jnp.int8)
    Wcls_pad = jnp.zeros((D, 128), jnp.float32).at[:, :NC].set(W_cls)
    eeT = edge_emb[:HD].T
    ee_blk = jax.scipy.linalg.block_diag(*([eeT] * H)).astype(jnp.float32)
    inv = jnp.float32(1.0 / math.sqrt(HD))
    W_all = jnp.concatenate(
        [Wq * inv, Wk, Wv, jnp.einsum('lde,ef->ldf', Wq, ee_blk) * inv],
        axis=2)

    out = _run_gat(h0, adj_i8, edge_types.astype(jnp.int32), W_all,
                   Wo, ln_scale, ln_bias, Wcls_pad)
    return out.reshape(B, 128)[:, :NC] + b_cls[None]


# speedup vs baseline: 1.1543x; 1.0145x over previous
import functools
import math

import jax
import jax.numpy as jnp
import numpy as np
from jax import lax
from jax.experimental import pallas as pl
from jax.experimental.pallas import tpu as pltpu

try:
    from jax.experimental.pallas import tpu_sc as plsc
    _HAS_SC = True
except ImportError:
    _HAS_SC = False

B, L, D, H, NL = 8, 512, 128, 8, 4
V = 100000
NET = 16
NC = 2
HD = D // H


def _pos_emb_np(length, dim):
    pos = np.arange(length)[:, None].astype(np.float64)
    i = np.arange(dim // 2)[None, :].astype(np.float64)
    ang = pos / np.power(10000.0, 2.0 * i / dim)
    pe = np.zeros((length, dim), dtype=np.float32)
    pe[:, 0::2] = np.sin(ang)
    pe[:, 1::2] = np.cos(ang)
    return pe


def _sc_gather(word_emb, flat_ids):
    info = plsc.get_sparse_core_info()
    ncores, nsub = info.num_cores, info.num_subcores
    nw = ncores * nsub
    total = B * L
    per_w = total // nw

    mesh = plsc.VectorSubcoreMesh(core_axis_name="c", subcore_axis_name="s")

    @functools.partial(
        pl.kernel,
        mesh=mesh,
        out_type=jax.ShapeDtypeStruct((total, D), jnp.float32),
        scratch_types=[
            pltpu.VMEM((per_w,), jnp.int32),
            pltpu.VMEM((per_w, D), jnp.float32),
            pltpu.SemaphoreType.DMA,
        ],
    )
    def gather_kernel(table_hbm, idx_hbm, out_hbm, idx_v, rows_v, sem):
        wid = lax.axis_index("s") * ncores + lax.axis_index("c")
        base = wid * per_w
        pltpu.sync_copy(idx_hbm.at[pl.ds(base, per_w)], idx_v)
        pltpu.async_copy(table_hbm.at[idx_v], rows_v, sem).wait()
        pltpu.sync_copy(rows_v, out_hbm.at[pl.ds(base, per_w)])

    return gather_kernel(word_emb, flat_ids)


def _gat_kernel(h0_ref, adj_ref, et_ref, wall_ref, wo_ref, lns_ref,
                lnb_ref, wcls_ref, cmat_ref, jmat_ref, out_ref,
                h_scr, o_scr):
    h_scr[...] = h0_ref[0]
    adj32 = adj_ref[0].astype(jnp.int32)
    etm = NET + adj32 * (et_ref[0] - NET)
    neg_col = jnp.full((L, HD), -1e9, jnp.float32)
    ones_col = jnp.ones((L, HD), jnp.float32)

    for l in range(NL):
        h = h_scr[...]
        qkvp = jnp.dot(h, wall_ref[l], preferred_element_type=jnp.float32)
        o_heads = []
        for hh in range(H):
            qh = qkvp[:, hh * HD:(hh + 1) * HD]
            kh = qkvp[:, D + hh * HD:D + (hh + 1) * HD]
            vh = qkvp[:, 2 * D + hh * HD:2 * D + (hh + 1) * HD]
            ph = qkvp[:, 3 * D + hh * HD:3 * D + (hh + 1) * HD]
            pf = jnp.concatenate([ph, neg_col], axis=1)
            scores = jax.lax.dot_general(
                qh.astype(jnp.bfloat16), kh.astype(jnp.bfloat16),
                (((1,), (1,)), ((), ())),
                preferred_element_type=jnp.float32)
            scores += jnp.take_along_axis(pf, etm, axis=1)
            e = jnp.exp(scores)
            va = jnp.concatenate([vh, ones_col], axis=1)
            oa = jnp.dot(e.astype(jnp.bfloat16), va.astype(jnp.bfloat16),
                         preferred_element_type=jnp.float32)
            o_heads.append(oa[:, :HD] / oa[:, HD:2 * HD])
        proj = jnp.dot(jnp.concatenate(o_heads, axis=1), wo_ref[l],
                       preferred_element_type=jnp.float32)
        x = h + proj
        xc = jnp.dot(x, cmat_ref[...], preferred_element_type=jnp.float32)
        v2 = jnp.dot(xc * xc, jmat_ref[...],
                     preferred_element_type=jnp.float32)
        h_scr[...] = xc * jax.lax.rsqrt(v2 + 1e-5) * lns_ref[l] \
            + lnb_ref[l]

    cls_h = h_scr[0:1, :]
    out_ref[0] = jnp.dot(cls_h, wcls_ref[...],
                         preferred_element_type=jnp.float32)


def _run_gat(h0, adj_i8, edge_types, W_all, Wo,
             ln_scale, ln_bias, Wcls_pad):
    rep2 = lambda b: (0, 0)
    rep3 = lambda b: (0, 0, 0)

    in_specs = [
            pl.BlockSpec((1, L, D), lambda b: (b, 0, 0)),
            pl.BlockSpec((1, L, L), lambda b: (b, 0, 0)),
            pl.BlockSpec((1, L, L), lambda b: (b, 0, 0)),
            pl.BlockSpec((NL, D, 4 * D), rep3),
            pl.BlockSpec((NL, D, D), rep3),
            pl.BlockSpec((NL, D), rep2),
            pl.BlockSpec((NL, D), rep2),
            pl.BlockSpec((D, 128), rep2),
            pl.BlockSpec((D, D), rep2),
            pl.BlockSpec((D, D), rep2),
    ]
    cmat = jnp.eye(D, dtype=jnp.float32) - 1.0 / D
    jmat = jnp.full((D, D), 1.0 / D, jnp.float32)
    return pl.pallas_call(
        _gat_kernel,
        grid=(B,),
        in_specs=in_specs,
        out_specs=pl.BlockSpec((1, 1, 128), lambda b: (b, 0, 0)),
        out_shape=jax.ShapeDtypeStruct((B, 1, 128), jnp.float32),
        scratch_shapes=[
            pltpu.VMEM((L, D), jnp.float32),
            pltpu.VMEM((L, D), jnp.float32),
        ],
    )(h0, adj_i8, edge_types, W_all, Wo,
      ln_scale, ln_bias, Wcls_pad, cmat, jmat)


def kernel(word_ids, batched_adj, edge_types, word_emb, edge_emb,
           Wq, Wk, Wv, Wo, ln_scale, ln_bias, W_cls, b_cls):
    flat_ids = word_ids.reshape(B * L).astype(jnp.int32)
    rows = _sc_gather(word_emb, flat_ids)
    pos = jnp.asarray(_pos_emb_np(L, D))
    h0 = rows.reshape(B, L, D) + pos[None]

    adj_i8 = batched_adj.astype(jnp.int8)
    Wcls_pad = jnp.zeros((D, 128), jnp.float32).at[:, :NC].set(W_cls)
    eeT = edge_emb[:HD].T
    ee_blk = jax.scipy.linalg.block_diag(*([eeT] * H)).astype(jnp.float32)
    inv = jnp.float32(1.0 / math.sqrt(HD))
    W_all = jnp.concatenate(
        [Wq * inv, Wk, Wv, jnp.einsum('lde,ef->ldf', Wq, ee_blk) * inv],
        axis=2)

    out = _run_gat(h0, adj_i8, edge_types.astype(jnp.int32), W_all,
                   Wo, ln_scale, ln_bias, Wcls_pad)
    return out.reshape(B, 128)[:, :NC] + b_cls[None]
